# baseline (speedup 1.0000x reference)
import jax
import jax.numpy as jnp
from jax import lax
from jax.experimental import pallas as pl

_K = 7
_NV = 100
_NP = 128
_J = 16
_BNS = float(1.0 / (1.0 + 1e-5) ** 0.5)


def _knn_onehots(d2t, k):
    cols = d2t.shape[1]
    row = lax.broadcasted_iota(jnp.int32, (_NP, cols), 0).astype(jnp.float32)
    big = jnp.float32(2.0 * _NP)
    ohs = []
    for _ in range(k):
        v, ix = d2t, row
        for half in (64, 32, 16, 8):
            a, b = v[:half], v[half:]
            c = a <= b
            v = jnp.minimum(a, b)
            ix = jnp.where(c, ix[:half], ix[half:])
        m = jnp.min(v, axis=0, keepdims=True)
        am = jnp.min(jnp.where(v <= m, ix, big), axis=0, keepdims=True)
        eqb = row == am
        ohs.append(jnp.where(eqb, jnp.float32(1.0),
                             jnp.float32(0.0)).astype(jnp.bfloat16))
        d2t = jnp.where(eqb, jnp.float32(1e10), d2t)
    return ohs


def _relu(x):
    return jnp.maximum(x, 0.0)


def _dot(a, b):
    return jnp.dot(a, b, preferred_element_type=jnp.float32)


def _dott(a, b):
    return lax.dot_general(a, b, (((1,), (1,)), ((), ())),
                           preferred_element_type=jnp.float32)


def _edge_conv(d2m, u, v, w2, b2, w3, b3, scv, nj, lowp):
    ohs = _knn_onehots(d2m, _K)
    if lowp:
        v = v.astype(jnp.bfloat16)
    h1 = []
    for j in range(nj):
        ohj = jnp.concatenate(
            [oh[:, j * _NP:(j + 1) * _NP] for oh in ohs], axis=1)
        vj = v[:, j * _NP:(j + 1) * _NP]
        nbr = lax.dot_general(vj, ohj, (((1,), (0,)), ((), ())),
                              preferred_element_type=jnp.float32)
        uj = u[:, j * _NP:(j + 1) * _NP]
        u7 = jnp.concatenate([uj] * _K, axis=1)
        h1.append(_relu(u7 + nbr))
    h1 = jnp.concatenate(h1, axis=1)
    if lowp:
        h1 = h1.astype(jnp.bfloat16)
    h2 = _relu(_dot(w2, h1) + b2)
    if lowp:
        h2 = h2.astype(jnp.bfloat16)
    h3 = _relu(_dot(w3, h2) + b3)
    parts = []
    for j in range(nj):
        base = j * _K * _NP
        agg = h3[:, base:base + _NP]
        for t in range(1, _K):
            agg = agg + h3[:, base + t * _NP:base + (t + 1) * _NP]
        parts.append(agg)
    agg = jnp.concatenate(parts, axis=1) * jnp.float32(1.0 / _K)
    return _relu(agg + scv)


def _pn_kernel(pts_ref, f_ref,
               wu1_ref, wv1_ref, b11_ref, w12_ref, b12_ref, w13_ref, b13_ref,
               wsc1_ref, bsc1_ref,
               wu2_ref, wv2_ref, b21_ref, w22_ref, b22_ref, w23_ref, b23_ref,
               wsc2_ref, bsc2_ref,
               wf1_ref, wf2_ref, bf_ref, out_ref):
    nj = f_ref.shape[0]
    nv = f_ref.shape[2]
    zf = jnp.zeros((f_ref.shape[1], _NP - nv), jnp.float32)
    zp = jnp.zeros((2, _NP - nv), jnp.float32)

    rowi = lax.broadcasted_iota(jnp.int32, (_NP, _NP), 0)
    coli = lax.broadcasted_iota(jnp.int32, (_NP, _NP), 1)
    diagm = jnp.where(rowi == coli, jnp.float32(1e10), jnp.float32(0.0))
    colmask = jnp.where(lax.broadcasted_iota(jnp.int32, (1, _NP), 1) >= _NV,
                        jnp.float32(1e10), jnp.float32(0.0))
    ones = jnp.ones((1, _NP), jnp.float32)

    fparts = []
    for j in range(nj):
        fparts += [f_ref[j], zf]
    f_all = jnp.concatenate(fparts, axis=1)

    d2s = []
    for j in range(nj):
        pos = jnp.concatenate([pts_ref[j], zp], axis=1)
        xr, yr = pos[0:1, :], pos[1:2, :]
        sqm = xr * xr + yr * yr + colmask
        a_aug = jnp.concatenate([pos, ones], axis=0)
        b_aug = jnp.concatenate([pos * jnp.float32(-2.0), sqm], axis=0)
        cr = lax.dot_general(b_aug, a_aug, (((0,), (0,)), ((), ())),
                             preferred_element_type=jnp.float32)
        d2s.append(cr + diagm)
    d2 = jnp.concatenate(d2s, axis=1)

    u1 = _dot(wu1_ref[...], f_all) + b11_ref[...]
    v1 = _dot(wv1_ref[...], f_all)
    sc1 = _dot(wsc1_ref[...], f_all) + bsc1_ref[...]
    x1 = _edge_conv(d2, u1, v1, w12_ref[...], b12_ref[...],
                    w13_ref[...], b13_ref[...], sc1, nj,
                    lowp=False)

    sq2 = (jnp.sum(x1 * x1, axis=0, keepdims=True)
           + jnp.concatenate([colmask] * nj, axis=1))
    d2s = []
    for j in range(nj):
        x1j = x1[:, j * _NP:(j + 1) * _NP]
        a_aug = jnp.concatenate([x1j, ones], axis=0)
        b_aug = jnp.concatenate([x1j * jnp.float32(-2.0),
                                 sq2[:, j * _NP:(j + 1) * _NP]], axis=0)
        cr = lax.dot_general(b_aug, a_aug, (((0,), (0,)), ((), ())),
                             preferred_element_type=jnp.float32)
        d2s.append(cr + diagm)
    d2b = jnp.concatenate(d2s, axis=1)
    x1b = x1.astype(jnp.bfloat16)
    u2 = _dot(wu2_ref[...], x1b) + b21_ref[...]
    v2 = _dot(wv2_ref[...], x1b)
    sc2 = _dot(wsc2_ref[...], x1b) + bsc2_ref[...]
    x2 = _edge_conv(d2b, u2, v2, w22_ref[...], b22_ref[...],
                    w23_ref[...], b23_ref[...], sc2, nj,
                    lowp=True)

    fts = _relu(_dot(wf1_ref[...], x1) + _dot(wf2_ref[...], x2)
                + bf_ref[...])
    for j in range(nj):
        pooled = jnp.sum(fts[:, j * _NP:j * _NP + _NV], axis=1)
        out_ref[j, 0, :] = pooled * jnp.float32(1.0 / _NV)


def _head_kernel(p_ref, w1_ref, b1_ref, w2_ref, b2_ref, out_ref):
    h = _relu(_dott(p_ref[...], w1_ref[...]) + b1_ref[...])
    out_ref[...] = _dott(h, w2_ref[...]) + b2_ref[...]


def _fold_edge_params(mlp, sc, in_scale, in_bias):
    s = _BNS
    (w1, b1, g1, e1), (w2, b2, g2, e2), (w3, b3, g3, e3) = mlp
    fin = w1.shape[1] // 2
    a = (g1 * s)[:, None] * (w1[:, :fin] - w1[:, fin:])
    bm = (g1 * s)[:, None] * w1[:, fin:]
    bias1 = (g1 * s) * b1 + e1
    wsc, gsc, bsc = sc
    wscf = (gsc * s)[:, None] * wsc
    bscf = bsc
    if in_scale is not None:
        bias1 = bias1 + a @ in_bias + bm @ in_bias
        bscf = bscf + (gsc * s) * (wsc @ in_bias)
        a = a * in_scale[None, :]
        bm = bm * in_scale[None, :]
        wscf = wscf * in_scale[None, :]
    w2f = (g2 * s)[:, None] * w2
    b2f = (g2 * s) * b2 + e2
    w3f = (g3 * s)[:, None] * w3
    b3f = (g3 * s) * b3 + e3
    return a, bm, bias1, w2f, b2f, w3f, b3f, wscf, bscf


def kernel(points, features, params):
    b, _, n = points.shape
    s = _BNS

    gb, bb = params['bn_fts']
    wu1, wv1, b11, w12, b12, w13, b13, wsc1, bsc1 = _fold_edge_params(
        params['conv1_mlp'], params['conv1_sc'], gb * s, bb)
    wu2, wv2, b21, w22, b22, w23, b23, wsc2, bsc2 = _fold_edge_params(
        params['conv2_mlp'], params['conv2_sc'], None, None)
    wf, gf, bf = params['fusion']
    wff = (gf * s)[:, None] * wf
    wf1, wf2 = wff[:, :32], wff[:, 32:]

    nf = features.shape[1]

    def col(x):
        return x[:, None]

    bcast2 = lambda shape: pl.BlockSpec(shape, lambda i: (0, 0))
    data3 = lambda shape: pl.BlockSpec(shape, lambda i: (i, 0, 0))

    pooled = pl.pallas_call(
        _pn_kernel,
        grid=(b // _J,),
        in_specs=[
            data3((_J, 2, n)),
            data3((_J, nf, n)),
            bcast2(wu1.shape), bcast2(wv1.shape), bcast2((32, 1)),
            bcast2(w12.shape), bcast2((32, 1)),
            bcast2(w13.shape), bcast2((32, 1)),
            bcast2(wsc1.shape), bcast2((32, 1)),
            bcast2(wu2.shape), bcast2(wv2.shape), bcast2((64, 1)),
            bcast2(w22.shape), bcast2((64, 1)),
            bcast2(w23.shape), bcast2((64, 1)),
            bcast2(wsc2.shape), bcast2((64, 1)),
            bcast2(wf1.shape), bcast2(wf2.shape), bcast2((128, 1)),
        ],
        out_specs=pl.BlockSpec((_J, 1, 128), lambda i: (i, 0, 0)),
        out_shape=jax.ShapeDtypeStruct((b, 1, 128), jnp.float32),
    )(points, features,
      wu1, wv1, col(b11), w12, col(b12), w13, col(b13), wsc1, col(bsc1),
      wu2.astype(jnp.bfloat16), wv2.astype(jnp.bfloat16), col(b21),
      w22.astype(jnp.bfloat16), col(b22),
      w23.astype(jnp.bfloat16), col(b23),
      wsc2.astype(jnp.bfloat16), col(bsc2),
      wf1, wf2, col(bf))
    pooled = pooled.reshape(b, 128)

    w1, b1 = params['fc1']
    w2, b2 = params['fc2']
    logits = pl.pallas_call(
        _head_kernel,
        grid=(1,),
        in_specs=[
            pl.BlockSpec((b, 128), lambda i: (0, 0)),
            pl.BlockSpec(w1.shape, lambda i: (0, 0)),
            pl.BlockSpec((1, 128), lambda i: (0, 0)),
            pl.BlockSpec(w2.shape, lambda i: (0, 0)),
            pl.BlockSpec((1, 5), lambda i: (0, 0)),
        ],
        out_specs=pl.BlockSpec((b, 5), lambda i: (0, 0)),
        out_shape=jax.ShapeDtypeStruct((b, 5), jnp.float32),
    )(pooled, w1, b1[None, :], w2, b2[None, :])
    return logits


# speedup vs baseline: 1.0065x; 1.0065x over previous
import jax
import jax.numpy as jnp
from jax import lax
from jax.experimental import pallas as pl

_K = 7
_NV = 100
_NP = 128
_J = 16
_BNS = float(1.0 / (1.0 + 1e-5) ** 0.5)


def _knn_onehots(d2t, k):
    cols = d2t.shape[1]
    row = lax.broadcasted_iota(jnp.int32, (_NP, cols), 0).astype(jnp.float32)
    big = jnp.float32(2.0 * _NP)
    ohs = []
    for t in range(k):
        v, ix = d2t, row
        for half in (64, 32, 16, 8):
            a, b = v[:half], v[half:]
            c = a <= b
            v = jnp.minimum(a, b)
            ix = jnp.where(c, ix[:half], ix[half:])
        m = jnp.min(v, axis=0, keepdims=True)
        am = jnp.min(jnp.where(v <= m, ix, big), axis=0, keepdims=True)
        eqb = row == am
        ohs.append(jnp.where(eqb, jnp.float32(1.0),
                             jnp.float32(0.0)).astype(jnp.bfloat16))
        if t < k - 1:
            d2t = jnp.where(eqb, jnp.float32(1e10), d2t)
    return ohs


def _relu(x):
    return jnp.maximum(x, 0.0)


def _dot(a, b):
    return jnp.dot(a, b, preferred_element_type=jnp.float32)


def _dott(a, b):
    return lax.dot_general(a, b, (((1,), (1,)), ((), ())),
                           preferred_element_type=jnp.float32)


def _edge_conv(d2m, u, v, w2, b2, w3, b3, scv, nj):
    ohs = _knn_onehots(d2m, _K)
    h1 = []
    for j in range(nj):
        ohj = jnp.concatenate(
            [oh[:, j * _NP:(j + 1) * _NP] for oh in ohs], axis=1)
        vj = v[:, j * _NP:(j + 1) * _NP]
        nbr = lax.dot_general(vj, ohj, (((1,), (0,)), ((), ())),
                              preferred_element_type=jnp.float32)
        uj = u[:, j * _NP:(j + 1) * _NP]
        u7 = jnp.concatenate([uj] * _K, axis=1)
        h1.append(_relu(u7 + nbr))
    h1 = jnp.concatenate(h1, axis=1)
    h2 = _relu(_dot(w2, h1) + b2)
    h3 = _relu(_dot(w3, h2) + b3)
    parts = []
    for j in range(nj):
        base = j * _K * _NP
        agg = h3[:, base:base + _NP]
        for t in range(1, _K):
            agg = agg + h3[:, base + t * _NP:base + (t + 1) * _NP]
        parts.append(agg)
    agg = jnp.concatenate(parts, axis=1) * jnp.float32(1.0 / _K)
    return _relu(agg + scv)


def _pn_kernel(pts_ref, f_ref,
               wu1_ref, wv1_ref, b11_ref, w12_ref, b12_ref, w13_ref, b13_ref,
               wsc1_ref, bsc1_ref,
               wu2_ref, wv2_ref, b21_ref, w22_ref, b22_ref, w23_ref, b23_ref,
               wsc2_ref, bsc2_ref,
               wf1_ref, wf2_ref, bf_ref, out_ref):
    nj = f_ref.shape[0]
    nv = f_ref.shape[2]
    zf = jnp.zeros((f_ref.shape[1], _NP - nv), jnp.float32)
    zp = jnp.zeros((2, _NP - nv), jnp.float32)

    rowi = lax.broadcasted_iota(jnp.int32, (_NP, _NP), 0)
    coli = lax.broadcasted_iota(jnp.int32, (_NP, _NP), 1)
    diagm = jnp.where(rowi == coli, jnp.float32(1e10), jnp.float32(0.0))
    colmask = jnp.where(lax.broadcasted_iota(jnp.int32, (1, _NP), 1) >= _NV,
                        jnp.float32(1e10), jnp.float32(0.0))
    ones = jnp.ones((1, _NP), jnp.float32)

    fparts = []
    for j in range(nj):
        fparts += [f_ref[j], zf]
    f_all = jnp.concatenate(fparts, axis=1)

    d2s = []
    for j in range(nj):
        pos = jnp.concatenate([pts_ref[j], zp], axis=1)
        xr, yr = pos[0:1, :], pos[1:2, :]
        sqm = xr * xr + yr * yr + colmask
        a_aug = jnp.concatenate([pos, ones], axis=0)
        b_aug = jnp.concatenate([pos * jnp.float32(-2.0), sqm], axis=0)
        cr = lax.dot_general(b_aug, a_aug, (((0,), (0,)), ((), ())),
                             preferred_element_type=jnp.float32)
        d2s.append(cr + diagm)
    d2 = jnp.concatenate(d2s, axis=1)

    u1 = _dot(wu1_ref[...], f_all) + b11_ref[...]
    v1 = _dot(wv1_ref[...], f_all)
    sc1 = _dot(wsc1_ref[...], f_all) + bsc1_ref[...]
    x1 = _edge_conv(d2, u1, v1, w12_ref[...], b12_ref[...],
                    w13_ref[...], b13_ref[...], sc1, nj)

    sq2 = (jnp.sum(x1 * x1, axis=0, keepdims=True)
           + jnp.concatenate([colmask] * nj, axis=1))
    d2s = []
    for j in range(nj):
        x1j = x1[:, j * _NP:(j + 1) * _NP]
        a_aug = jnp.concatenate([x1j, ones], axis=0)
        b_aug = jnp.concatenate([x1j * jnp.float32(-2.0),
                                 sq2[:, j * _NP:(j + 1) * _NP]], axis=0)
        cr = lax.dot_general(b_aug, a_aug, (((0,), (0,)), ((), ())),
                             preferred_element_type=jnp.float32)
        d2s.append(cr + diagm)
    d2b = jnp.concatenate(d2s, axis=1)
    u2 = _dot(wu2_ref[...], x1) + b21_ref[...]
    v2 = _dot(wv2_ref[...], x1)
    sc2 = _dot(wsc2_ref[...], x1) + bsc2_ref[...]
    x2 = _edge_conv(d2b, u2, v2, w22_ref[...], b22_ref[...],
                    w23_ref[...], b23_ref[...], sc2, nj)

    fts = _relu(_dot(wf1_ref[...], x1) + _dot(wf2_ref[...], x2)
                + bf_ref[...])
    for j in range(nj):
        pooled = jnp.sum(fts[:, j * _NP:j * _NP + _NV], axis=1)
        out_ref[j, 0, :] = pooled * jnp.float32(1.0 / _NV)


def _head_kernel(p_ref, w1_ref, b1_ref, w2_ref, b2_ref, out_ref):
    h = _relu(_dott(p_ref[...], w1_ref[...]) + b1_ref[...])
    out_ref[...] = _dott(h, w2_ref[...]) + b2_ref[...]


def _fold_edge_params(mlp, sc, in_scale, in_bias):
    s = _BNS
    (w1, b1, g1, e1), (w2, b2, g2, e2), (w3, b3, g3, e3) = mlp
    fin = w1.shape[1] // 2
    a = (g1 * s)[:, None] * (w1[:, :fin] - w1[:, fin:])
    bm = (g1 * s)[:, None] * w1[:, fin:]
    bias1 = (g1 * s) * b1 + e1
    wsc, gsc, bsc = sc
    wscf = (gsc * s)[:, None] * wsc
    bscf = bsc
    if in_scale is not None:
        bias1 = bias1 + a @ in_bias + bm @ in_bias
        bscf = bscf + (gsc * s) * (wsc @ in_bias)
        a = a * in_scale[None, :]
        bm = bm * in_scale[None, :]
        wscf = wscf * in_scale[None, :]
    w2f = (g2 * s)[:, None] * w2
    b2f = (g2 * s) * b2 + e2
    w3f = (g3 * s)[:, None] * w3
    b3f = (g3 * s) * b3 + e3
    return a, bm, bias1, w2f, b2f, w3f, b3f, wscf, bscf


def kernel(points, features, params):
    b, _, n = points.shape
    s = _BNS

    gb, bb = params['bn_fts']
    wu1, wv1, b11, w12, b12, w13, b13, wsc1, bsc1 = _fold_edge_params(
        params['conv1_mlp'], params['conv1_sc'], gb * s, bb)
    wu2, wv2, b21, w22, b22, w23, b23, wsc2, bsc2 = _fold_edge_params(
        params['conv2_mlp'], params['conv2_sc'], None, None)
    wf, gf, bf = params['fusion']
    wff = (gf * s)[:, None] * wf
    wf1, wf2 = wff[:, :32], wff[:, 32:]

    nf = features.shape[1]

    def col(x):
        return x[:, None]

    bcast2 = lambda shape: pl.BlockSpec(shape, lambda i: (0, 0))
    data3 = lambda shape: pl.BlockSpec(shape, lambda i: (i, 0, 0))

    pooled = pl.pallas_call(
        _pn_kernel,
        grid=(b // _J,),
        in_specs=[
            data3((_J, 2, n)),
            data3((_J, nf, n)),
            bcast2(wu1.shape), bcast2(wv1.shape), bcast2((32, 1)),
            bcast2(w12.shape), bcast2((32, 1)),
            bcast2(w13.shape), bcast2((32, 1)),
            bcast2(wsc1.shape), bcast2((32, 1)),
            bcast2(wu2.shape), bcast2(wv2.shape), bcast2((64, 1)),
            bcast2(w22.shape), bcast2((64, 1)),
            bcast2(w23.shape), bcast2((64, 1)),
            bcast2(wsc2.shape), bcast2((64, 1)),
            bcast2(wf1.shape), bcast2(wf2.shape), bcast2((128, 1)),
        ],
        out_specs=pl.BlockSpec((_J, 1, 128), lambda i: (i, 0, 0)),
        out_shape=jax.ShapeDtypeStruct((b, 1, 128), jnp.float32),
    )(points, features,
      wu1, wv1, col(b11), w12, col(b12), w13, col(b13), wsc1, col(bsc1),
      wu2, wv2, col(b21), w22, col(b22), w23, col(b23), wsc2, col(bsc2),
      wf1, wf2, col(bf))
    pooled = pooled.reshape(b, 128)

    w1, b1 = params['fc1']
    w2, b2 = params['fc2']
    logits = pl.pallas_call(
        _head_kernel,
        grid=(1,),
        in_specs=[
            pl.BlockSpec((b, 128), lambda i: (0, 0)),
            pl.BlockSpec(w1.shape, lambda i: (0, 0)),
            pl.BlockSpec((1, 128), lambda i: (0, 0)),
            pl.BlockSpec(w2.shape, lambda i: (0, 0)),
            pl.BlockSpec((1, 5), lambda i: (0, 0)),
        ],
        out_specs=pl.BlockSpec((b, 5), lambda i: (0, 0)),
        out_shape=jax.ShapeDtypeStruct((b, 5), jnp.float32),
    )(pooled, w1, b1[None, :], w2, b2[None, :])
    return logits


# speedup vs baseline: 1.0366x; 1.0299x over previous
import jax
import jax.numpy as jnp
from jax import lax
from jax.experimental import pallas as pl

_K = 7
_NV = 100
_NP = 128
_J = 32
_BNS = float(1.0 / (1.0 + 1e-5) ** 0.5)


def _knn_onehots(d2t, k):
    cols = d2t.shape[1]
    row = lax.broadcasted_iota(jnp.int32, (_NP, cols), 0).astype(jnp.float32)
    big = jnp.float32(2.0 * _NP)
    ohs = []
    for t in range(k):
        v, ix = d2t, row
        for half in (64, 32, 16, 8):
            a, b = v[:half], v[half:]
            c = a <= b
            v = jnp.minimum(a, b)
            ix = jnp.where(c, ix[:half], ix[half:])
        m = jnp.min(v, axis=0, keepdims=True)
        am = jnp.min(jnp.where(v <= m, ix, big), axis=0, keepdims=True)
        eqb = row == am
        ohs.append(jnp.where(eqb, jnp.float32(1.0),
                             jnp.float32(0.0)).astype(jnp.bfloat16))
        if t < k - 1:
            d2t = jnp.where(eqb, jnp.float32(1e10), d2t)
    return ohs


def _relu(x):
    return jnp.maximum(x, 0.0)


def _dot(a, b):
    return jnp.dot(a, b, preferred_element_type=jnp.float32)


def _dott(a, b):
    return lax.dot_general(a, b, (((1,), (1,)), ((), ())),
                           preferred_element_type=jnp.float32)


def _edge_conv(d2m, u, v, w2, b2, w3, b3, scv, nj):
    ohs = _knn_onehots(d2m, _K)
    h1 = []
    for j in range(nj):
        ohj = jnp.concatenate(
            [oh[:, j * _NP:(j + 1) * _NP] for oh in ohs], axis=1)
        vj = v[:, j * _NP:(j + 1) * _NP]
        nbr = lax.dot_general(vj, ohj, (((1,), (0,)), ((), ())),
                              preferred_element_type=jnp.float32)
        uj = u[:, j * _NP:(j + 1) * _NP]
        u7 = jnp.concatenate([uj] * _K, axis=1)
        h1.append(_relu(u7 + nbr))
    h1 = jnp.concatenate(h1, axis=1)
    h2 = _relu(_dot(w2, h1) + b2)
    h3 = _relu(_dot(w3, h2) + b3)
    parts = []
    for j in range(nj):
        base = j * _K * _NP
        agg = h3[:, base:base + _NP]
        for t in range(1, _K):
            agg = agg + h3[:, base + t * _NP:base + (t + 1) * _NP]
        parts.append(agg)
    agg = jnp.concatenate(parts, axis=1) * jnp.float32(1.0 / _K)
    return _relu(agg + scv)


def _pn_kernel(pts_ref, f_ref,
               wu1_ref, wv1_ref, b11_ref, w12_ref, b12_ref, w13_ref, b13_ref,
               wsc1_ref, bsc1_ref,
               wu2_ref, wv2_ref, b21_ref, w22_ref, b22_ref, w23_ref, b23_ref,
               wsc2_ref, bsc2_ref,
               wf1_ref, wf2_ref, bf_ref, out_ref):
    nj = f_ref.shape[0]
    nv = f_ref.shape[2]
    zf = jnp.zeros((f_ref.shape[1], _NP - nv), jnp.float32)
    zp = jnp.zeros((2, _NP - nv), jnp.float32)

    rowi = lax.broadcasted_iota(jnp.int32, (_NP, _NP), 0)
    coli = lax.broadcasted_iota(jnp.int32, (_NP, _NP), 1)
    diagm = jnp.where(rowi == coli, jnp.float32(1e10), jnp.float32(0.0))
    colmask = jnp.where(lax.broadcasted_iota(jnp.int32, (1, _NP), 1) >= _NV,
                        jnp.float32(1e10), jnp.float32(0.0))
    ones = jnp.ones((1, _NP), jnp.float32)

    fparts = []
    for j in range(nj):
        fparts += [f_ref[j], zf]
    f_all = jnp.concatenate(fparts, axis=1)

    d2s = []
    for j in range(nj):
        pos = jnp.concatenate([pts_ref[j], zp], axis=1)
        xr, yr = pos[0:1, :], pos[1:2, :]
        sqm = xr * xr + yr * yr + colmask
        a_aug = jnp.concatenate([pos, ones], axis=0)
        b_aug = jnp.concatenate([pos * jnp.float32(-2.0), sqm], axis=0)
        cr = lax.dot_general(b_aug, a_aug, (((0,), (0,)), ((), ())),
                             preferred_element_type=jnp.float32)
        d2s.append(cr + diagm)
    d2 = jnp.concatenate(d2s, axis=1)

    u1 = _dot(wu1_ref[...], f_all) + b11_ref[...]
    v1 = _dot(wv1_ref[...], f_all)
    sc1 = _dot(wsc1_ref[...], f_all) + bsc1_ref[...]
    x1 = _edge_conv(d2, u1, v1, w12_ref[...], b12_ref[...],
                    w13_ref[...], b13_ref[...], sc1, nj)

    sq2 = (jnp.sum(x1 * x1, axis=0, keepdims=True)
           + jnp.concatenate([colmask] * nj, axis=1))
    d2s = []
    for j in range(nj):
        x1j = x1[:, j * _NP:(j + 1) * _NP]
        a_aug = jnp.concatenate([x1j, ones], axis=0)
        b_aug = jnp.concatenate([x1j * jnp.float32(-2.0),
                                 sq2[:, j * _NP:(j + 1) * _NP]], axis=0)
        cr = lax.dot_general(b_aug, a_aug, (((0,), (0,)), ((), ())),
                             preferred_element_type=jnp.float32)
        d2s.append(cr + diagm)
    d2b = jnp.concatenate(d2s, axis=1)
    u2 = _dot(wu2_ref[...], x1) + b21_ref[...]
    v2 = _dot(wv2_ref[...], x1)
    sc2 = _dot(wsc2_ref[...], x1) + bsc2_ref[...]
    x2 = _edge_conv(d2b, u2, v2, w22_ref[...], b22_ref[...],
                    w23_ref[...], b23_ref[...], sc2, nj)

    fts = _relu(_dot(wf1_ref[...], x1) + _dot(wf2_ref[...], x2)
                + bf_ref[...])
    for j in range(nj):
        pooled = jnp.sum(fts[:, j * _NP:j * _NP + _NV], axis=1)
        out_ref[j, 0, :] = pooled * jnp.float32(1.0 / _NV)


def _head_kernel(p_ref, w1_ref, b1_ref, w2_ref, b2_ref, out_ref):
    h = _relu(_dott(p_ref[...], w1_ref[...]) + b1_ref[...])
    out_ref[...] = _dott(h, w2_ref[...]) + b2_ref[...]


def _fold_edge_params(mlp, sc, in_scale, in_bias):
    s = _BNS
    (w1, b1, g1, e1), (w2, b2, g2, e2), (w3, b3, g3, e3) = mlp
    fin = w1.shape[1] // 2
    a = (g1 * s)[:, None] * (w1[:, :fin] - w1[:, fin:])
    bm = (g1 * s)[:, None] * w1[:, fin:]
    bias1 = (g1 * s) * b1 + e1
    wsc, gsc, bsc = sc
    wscf = (gsc * s)[:, None] * wsc
    bscf = bsc
    if in_scale is not None:
        bias1 = bias1 + a @ in_bias + bm @ in_bias
        bscf = bscf + (gsc * s) * (wsc @ in_bias)
        a = a * in_scale[None, :]
        bm = bm * in_scale[None, :]
        wscf = wscf * in_scale[None, :]
    w2f = (g2 * s)[:, None] * w2
    b2f = (g2 * s) * b2 + e2
    w3f = (g3 * s)[:, None] * w3
    b3f = (g3 * s) * b3 + e3
    return a, bm, bias1, w2f, b2f, w3f, b3f, wscf, bscf


def kernel(points, features, params):
    b, _, n = points.shape
    s = _BNS

    gb, bb = params['bn_fts']
    wu1, wv1, b11, w12, b12, w13, b13, wsc1, bsc1 = _fold_edge_params(
        params['conv1_mlp'], params['conv1_sc'], gb * s, bb)
    wu2, wv2, b21, w22, b22, w23, b23, wsc2, bsc2 = _fold_edge_params(
        params['conv2_mlp'], params['conv2_sc'], None, None)
    wf, gf, bf = params['fusion']
    wff = (gf * s)[:, None] * wf
    wf1, wf2 = wff[:, :32], wff[:, 32:]

    nf = features.shape[1]

    def col(x):
        return x[:, None]

    bcast2 = lambda shape: pl.BlockSpec(shape, lambda i: (0, 0))
    data3 = lambda shape: pl.BlockSpec(shape, lambda i: (i, 0, 0))

    pooled = pl.pallas_call(
        _pn_kernel,
        grid=(b // _J,),
        in_specs=[
            data3((_J, 2, n)),
            data3((_J, nf, n)),
            bcast2(wu1.shape), bcast2(wv1.shape), bcast2((32, 1)),
            bcast2(w12.shape), bcast2((32, 1)),
            bcast2(w13.shape), bcast2((32, 1)),
            bcast2(wsc1.shape), bcast2((32, 1)),
            bcast2(wu2.shape), bcast2(wv2.shape), bcast2((64, 1)),
            bcast2(w22.shape), bcast2((64, 1)),
            bcast2(w23.shape), bcast2((64, 1)),
            bcast2(wsc2.shape), bcast2((64, 1)),
            bcast2(wf1.shape), bcast2(wf2.shape), bcast2((128, 1)),
        ],
        out_specs=pl.BlockSpec((_J, 1, 128), lambda i: (i, 0, 0)),
        out_shape=jax.ShapeDtypeStruct((b, 1, 128), jnp.float32),
    )(points, features,
      wu1, wv1, col(b11), w12, col(b12), w13, col(b13), wsc1, col(bsc1),
      wu2, wv2, col(b21), w22, col(b22), w23, col(b23), wsc2, col(bsc2),
      wf1, wf2, col(bf))
    pooled = pooled.reshape(b, 128)

    w1, b1 = params['fc1']
    w2, b2 = params['fc2']
    logits = pl.pallas_call(
        _head_kernel,
        grid=(1,),
        in_specs=[
            pl.BlockSpec((b, 128), lambda i: (0, 0)),
            pl.BlockSpec(w1.shape, lambda i: (0, 0)),
            pl.BlockSpec((1, 128), lambda i: (0, 0)),
            pl.BlockSpec(w2.shape, lambda i: (0, 0)),
            pl.BlockSpec((1, 5), lambda i: (0, 0)),
        ],
        out_specs=pl.BlockSpec((b, 5), lambda i: (0, 0)),
        out_shape=jax.ShapeDtypeStruct((b, 5), jnp.float32),
    )(pooled, w1, b1[None, :], w2, b2[None, :])
    return logits


# speedup vs baseline: 1.0445x; 1.0075x over previous
import jax
import jax.numpy as jnp
from jax import lax
from jax.experimental import pallas as pl

_K = 7
_NV = 100
_NP = 128
_J = 64
_BNS = float(1.0 / (1.0 + 1e-5) ** 0.5)


def _knn_onehots(d2t, k):
    cols = d2t.shape[1]
    row = lax.broadcasted_iota(jnp.int32, (_NP, cols), 0).astype(jnp.float32)
    big = jnp.float32(2.0 * _NP)
    ohs = []
    for t in range(k):
        v, ix = d2t, row
        for half in (64, 32, 16, 8):
            a, b = v[:half], v[half:]
            c = a <= b
            v = jnp.minimum(a, b)
            ix = jnp.where(c, ix[:half], ix[half:])
        m = jnp.min(v, axis=0, keepdims=True)
        am = jnp.min(jnp.where(v <= m, ix, big), axis=0, keepdims=True)
        eqb = row == am
        ohs.append(jnp.where(eqb, jnp.float32(1.0),
                             jnp.float32(0.0)).astype(jnp.bfloat16))
        if t < k - 1:
            d2t = jnp.where(eqb, jnp.float32(1e10), d2t)
    return ohs


def _relu(x):
    return jnp.maximum(x, 0.0)


def _dot(a, b):
    return jnp.dot(a, b, preferred_element_type=jnp.float32)


def _dott(a, b):
    return lax.dot_general(a, b, (((1,), (1,)), ((), ())),
                           preferred_element_type=jnp.float32)


def _edge_conv(d2m, u, v, w2, b2, w3, b3, scv, nj):
    ohs = _knn_onehots(d2m, _K)
    h1 = []
    for j in range(nj):
        ohj = jnp.concatenate(
            [oh[:, j * _NP:(j + 1) * _NP] for oh in ohs], axis=1)
        vj = v[:, j * _NP:(j + 1) * _NP]
        nbr = lax.dot_general(vj, ohj, (((1,), (0,)), ((), ())),
                              preferred_element_type=jnp.float32)
        uj = u[:, j * _NP:(j + 1) * _NP]
        u7 = jnp.concatenate([uj] * _K, axis=1)
        h1.append(_relu(u7 + nbr))
    h1 = jnp.concatenate(h1, axis=1)
    h2 = _relu(_dot(w2, h1) + b2)
    h3 = _relu(_dot(w3, h2) + b3)
    parts = []
    for j in range(nj):
        base = j * _K * _NP
        agg = h3[:, base:base + _NP]
        for t in range(1, _K):
            agg = agg + h3[:, base + t * _NP:base + (t + 1) * _NP]
        parts.append(agg)
    agg = jnp.concatenate(parts, axis=1) * jnp.float32(1.0 / _K)
    return _relu(agg + scv)


def _pn_kernel(pts_ref, f_ref,
               wu1_ref, wv1_ref, b11_ref, w12_ref, b12_ref, w13_ref, b13_ref,
               wsc1_ref, bsc1_ref,
               wu2_ref, wv2_ref, b21_ref, w22_ref, b22_ref, w23_ref, b23_ref,
               wsc2_ref, bsc2_ref,
               wf1_ref, wf2_ref, bf_ref, out_ref):
    nj = f_ref.shape[0]
    nv = f_ref.shape[2]
    zf = jnp.zeros((f_ref.shape[1], _NP - nv), jnp.float32)
    zp = jnp.zeros((2, _NP - nv), jnp.float32)

    rowi = lax.broadcasted_iota(jnp.int32, (_NP, _NP), 0)
    coli = lax.broadcasted_iota(jnp.int32, (_NP, _NP), 1)
    diagm = jnp.where(rowi == coli, jnp.float32(1e10), jnp.float32(0.0))
    colmask = jnp.where(lax.broadcasted_iota(jnp.int32, (1, _NP), 1) >= _NV,
                        jnp.float32(1e10), jnp.float32(0.0))
    ones = jnp.ones((1, _NP), jnp.float32)

    fparts = []
    for j in range(nj):
        fparts += [f_ref[j], zf]
    f_all = jnp.concatenate(fparts, axis=1)

    d2s = []
    for j in range(nj):
        pos = jnp.concatenate([pts_ref[j], zp], axis=1)
        xr, yr = pos[0:1, :], pos[1:2, :]
        sqm = xr * xr + yr * yr + colmask
        a_aug = jnp.concatenate([pos, ones], axis=0)
        b_aug = jnp.concatenate([pos * jnp.float32(-2.0), sqm], axis=0)
        cr = lax.dot_general(b_aug, a_aug, (((0,), (0,)), ((), ())),
                             preferred_element_type=jnp.float32)
        d2s.append(cr + diagm)
    d2 = jnp.concatenate(d2s, axis=1)

    u1 = _dot(wu1_ref[...], f_all) + b11_ref[...]
    v1 = _dot(wv1_ref[...], f_all)
    sc1 = _dot(wsc1_ref[...], f_all) + bsc1_ref[...]
    x1 = _edge_conv(d2, u1, v1, w12_ref[...], b12_ref[...],
                    w13_ref[...], b13_ref[...], sc1, nj)

    sq2 = (jnp.sum(x1 * x1, axis=0, keepdims=True)
           + jnp.concatenate([colmask] * nj, axis=1))
    d2s = []
    for j in range(nj):
        x1j = x1[:, j * _NP:(j + 1) * _NP]
        a_aug = jnp.concatenate([x1j, ones], axis=0)
        b_aug = jnp.concatenate([x1j * jnp.float32(-2.0),
                                 sq2[:, j * _NP:(j + 1) * _NP]], axis=0)
        cr = lax.dot_general(b_aug, a_aug, (((0,), (0,)), ((), ())),
                             preferred_element_type=jnp.float32)
        d2s.append(cr + diagm)
    d2b = jnp.concatenate(d2s, axis=1)
    u2 = _dot(wu2_ref[...], x1) + b21_ref[...]
    v2 = _dot(wv2_ref[...], x1)
    sc2 = _dot(wsc2_ref[...], x1) + bsc2_ref[...]
    x2 = _edge_conv(d2b, u2, v2, w22_ref[...], b22_ref[...],
                    w23_ref[...], b23_ref[...], sc2, nj)

    fts = _relu(_dot(wf1_ref[...], x1) + _dot(wf2_ref[...], x2)
                + bf_ref[...])
    for j in range(nj):
        pooled = jnp.sum(fts[:, j * _NP:j * _NP + _NV], axis=1)
        out_ref[j, 0, :] = pooled * jnp.float32(1.0 / _NV)


def _head_kernel(p_ref, w1_ref, b1_ref, w2_ref, b2_ref, out_ref):
    h = _relu(_dott(p_ref[...], w1_ref[...]) + b1_ref[...])
    out_ref[...] = _dott(h, w2_ref[...]) + b2_ref[...]


def _fold_edge_params(mlp, sc, in_scale, in_bias):
    s = _BNS
    (w1, b1, g1, e1), (w2, b2, g2, e2), (w3, b3, g3, e3) = mlp
    fin = w1.shape[1] // 2
    a = (g1 * s)[:, None] * (w1[:, :fin] - w1[:, fin:])
    bm = (g1 * s)[:, None] * w1[:, fin:]
    bias1 = (g1 * s) * b1 + e1
    wsc, gsc, bsc = sc
    wscf = (gsc * s)[:, None] * wsc
    bscf = bsc
    if in_scale is not None:
        bias1 = bias1 + a @ in_bias + bm @ in_bias
        bscf = bscf + (gsc * s) * (wsc @ in_bias)
        a = a * in_scale[None, :]
        bm = bm * in_scale[None, :]
        wscf = wscf * in_scale[None, :]
    w2f = (g2 * s)[:, None] * w2
    b2f = (g2 * s) * b2 + e2
    w3f = (g3 * s)[:, None] * w3
    b3f = (g3 * s) * b3 + e3
    return a, bm, bias1, w2f, b2f, w3f, b3f, wscf, bscf


def kernel(points, features, params):
    b, _, n = points.shape
    s = _BNS

    gb, bb = params['bn_fts']
    wu1, wv1, b11, w12, b12, w13, b13, wsc1, bsc1 = _fold_edge_params(
        params['conv1_mlp'], params['conv1_sc'], gb * s, bb)
    wu2, wv2, b21, w22, b22, w23, b23, wsc2, bsc2 = _fold_edge_params(
        params['conv2_mlp'], params['conv2_sc'], None, None)
    wf, gf, bf = params['fusion']
    wff = (gf * s)[:, None] * wf
    wf1, wf2 = wff[:, :32], wff[:, 32:]

    nf = features.shape[1]

    def col(x):
        return x[:, None]

    bcast2 = lambda shape: pl.BlockSpec(shape, lambda i: (0, 0))
    data3 = lambda shape: pl.BlockSpec(shape, lambda i: (i, 0, 0))

    pooled = pl.pallas_call(
        _pn_kernel,
        grid=(b // _J,),
        in_specs=[
            data3((_J, 2, n)),
            data3((_J, nf, n)),
            bcast2(wu1.shape), bcast2(wv1.shape), bcast2((32, 1)),
            bcast2(w12.shape), bcast2((32, 1)),
            bcast2(w13.shape), bcast2((32, 1)),
            bcast2(wsc1.shape), bcast2((32, 1)),
            bcast2(wu2.shape), bcast2(wv2.shape), bcast2((64, 1)),
            bcast2(w22.shape), bcast2((64, 1)),
            bcast2(w23.shape), bcast2((64, 1)),
            bcast2(wsc2.shape), bcast2((64, 1)),
            bcast2(wf1.shape), bcast2(wf2.shape), bcast2((128, 1)),
        ],
        out_specs=pl.BlockSpec((_J, 1, 128), lambda i: (i, 0, 0)),
        out_shape=jax.ShapeDtypeStruct((b, 1, 128), jnp.float32),
    )(points, features,
      wu1, wv1, col(b11), w12, col(b12), w13, col(b13), wsc1, col(bsc1),
      wu2, wv2, col(b21), w22, col(b22), w23, col(b23), wsc2, col(bsc2),
      wf1, wf2, col(bf))
    pooled = pooled.reshape(b, 128)

    w1, b1 = params['fc1']
    w2, b2 = params['fc2']
    logits = pl.pallas_call(
        _head_kernel,
        grid=(1,),
        in_specs=[
            pl.BlockSpec((b, 128), lambda i: (0, 0)),
            pl.BlockSpec(w1.shape, lambda i: (0, 0)),
            pl.BlockSpec((1, 128), lambda i: (0, 0)),
            pl.BlockSpec(w2.shape, lambda i: (0, 0)),
            pl.BlockSpec((1, 5), lambda i: (0, 0)),
        ],
        out_specs=pl.BlockSpec((b, 5), lambda i: (0, 0)),
        out_shape=jax.ShapeDtypeStruct((b, 5), jnp.float32),
    )(pooled, w1, b1[None, :], w2, b2[None, :])
    return logits
